# baseline (speedup 1.0000x reference)
import jax
import jax.numpy as jnp
from jax import lax
from jax.experimental import pallas as pl
from jax.experimental.pallas import tpu as pltpu

_TAU = 0.07
_Q, _K, _D = 2048, 16384, 256
_BQ = 256
_BK = 2048
_NQ = _Q // _BQ
_NK = _K // _BK


def _fused_body(y_ref, x_ref, o_ref, m_ref, s_ref):
    k = pl.program_id(1)
    y = y_ref[...]
    x = x_ref[...]
    dot = lax.dot_general(
        y, x, (((1,), (1,)), ((), ())),
        preferred_element_type=jnp.float32,
        precision=lax.Precision.DEFAULT,
    )
    sqx = lax.dot_general(
        jnp.ones((1, _D), jnp.float32), x * x, (((1,), (1,)), ((), ())),
        preferred_element_type=jnp.float32,
        precision=lax.Precision.HIGHEST,
    )
    logits = (2.0 * dot - sqx) * (1.0 / _TAU)

    m_c = jnp.max(logits, axis=1, keepdims=True)
    e = jnp.exp(logits - m_c)
    s_c = jnp.sum(e, axis=1, keepdims=True)

    o_ref[:, pl.ds(k * _BK, _BK)] = e
    m_ref[k] = jnp.broadcast_to(m_c, (_BQ, 128))
    s_ref[k] = jnp.broadcast_to(s_c, (_BQ, 128))

    @pl.when(k == _NK - 1)
    def _finalize():
        m_all = m_ref[...]
        s_all = s_ref[...]
        m = jnp.max(m_all, axis=0)
        w = jnp.exp(m_all - m)
        s = jnp.sum(s_all * w, axis=0)
        r = w / s
        for c in range(_NK):
            sl = pl.ds(c * _BK, _BK)
            o_ref[:, sl] = o_ref[:, sl] * r[c, :, 0:1]


def kernel(X, Y):
    return pl.pallas_call(
        _fused_body,
        grid=(_NQ, _NK),
        in_specs=[
            pl.BlockSpec((_BQ, _D), lambda q, k: (q, 0)),
            pl.BlockSpec((_BK, _D), lambda q, k: (k, 0)),
        ],
        out_specs=pl.BlockSpec((_BQ, _K), lambda q, k: (q, 0)),
        out_shape=jax.ShapeDtypeStruct((_Q, _K), jnp.float32),
        scratch_shapes=[
            pltpu.VMEM((_NK, _BQ, 128), jnp.float32),
            pltpu.VMEM((_NK, _BQ, 128), jnp.float32),
        ],
        compiler_params=pltpu.CompilerParams(
            dimension_semantics=("parallel", "arbitrary"),
        ),
    )(Y, X)


# speedup vs baseline: 1.0001x; 1.0001x over previous
import jax
import jax.numpy as jnp
from jax import lax
from jax.experimental import pallas as pl
from jax.experimental.pallas import tpu as pltpu

_TAU = 0.07
_Q, _K, _D = 2048, 16384, 256
_BQ = 256
_BK = 2048
_NQ = _Q // _BQ
_NK = _K // _BK


def _fused_body(y_ref, x_ref, o_ref, m_ref, s_ref):
    k = pl.program_id(1)
    y = y_ref[...]
    x = x_ref[...]
    dot = lax.dot_general(
        y.astype(jnp.bfloat16), x.astype(jnp.bfloat16), (((1,), (1,)), ((), ())),
        preferred_element_type=jnp.float32,
        precision=lax.Precision.DEFAULT,
    )
    sqx = lax.dot_general(
        jnp.ones((1, _D), jnp.float32), x * x, (((1,), (1,)), ((), ())),
        preferred_element_type=jnp.float32,
        precision=lax.Precision.HIGHEST,
    )
    logits = (2.0 * dot - sqx) * (1.0 / _TAU)

    m_c = jnp.max(logits, axis=1, keepdims=True)
    e = jnp.exp(logits - m_c)
    s_c = jnp.sum(e, axis=1, keepdims=True)

    o_ref[:, pl.ds(k * _BK, _BK)] = e
    m_ref[k] = jnp.broadcast_to(m_c, (_BQ, 128))
    s_ref[k] = jnp.broadcast_to(s_c, (_BQ, 128))

    @pl.when(k == _NK - 1)
    def _finalize():
        m_all = m_ref[...]
        s_all = s_ref[...]
        m = jnp.max(m_all, axis=0)
        w = jnp.exp(m_all - m)
        s = jnp.sum(s_all * w, axis=0)
        r = w / s
        for c in range(_NK):
            sl = pl.ds(c * _BK, _BK)
            o_ref[:, sl] = o_ref[:, sl] * r[c, :, 0:1]


def kernel(X, Y):
    return pl.pallas_call(
        _fused_body,
        grid=(_NQ, _NK),
        in_specs=[
            pl.BlockSpec((_BQ, _D), lambda q, k: (q, 0)),
            pl.BlockSpec((_BK, _D), lambda q, k: (k, 0)),
        ],
        out_specs=pl.BlockSpec((_BQ, _K), lambda q, k: (q, 0)),
        out_shape=jax.ShapeDtypeStruct((_Q, _K), jnp.float32),
        scratch_shapes=[
            pltpu.VMEM((_NK, _BQ, 128), jnp.float32),
            pltpu.VMEM((_NK, _BQ, 128), jnp.float32),
        ],
        compiler_params=pltpu.CompilerParams(
            dimension_semantics=("parallel", "arbitrary"),
        ),
    )(Y, X)
